# baseline (speedup 1.0000x reference)
import functools

import jax
import jax.numpy as jnp
from jax import lax
from jax.experimental import pallas as pl
from jax.experimental.pallas import tpu as pltpu
from jax.experimental.pallas import tpu_sc as plsc

_NC = 2
_NS = 16
_LANES = 16
_NBUF = 2


def kernel(pitch, gd, vel, P_table, Wgd, bgd, Wv, bv, pos_table):
    B, T = pitch.shape
    NP = P_table.shape[0]
    D_P = P_table.shape[1]
    D_GD = Wgd.shape[1]
    D_V = Wv.shape[1]
    D = D_P + D_GD + D_V

    A = jnp.concatenate(
        [
            pos_table[:, :D_P],
            pos_table[:, D_P:] + bgd[None, :],
            jnp.broadcast_to(bv[None, :], (T, D_V)),
        ],
        axis=1,
    )

    NW = _NC * _NS
    RPW = B // NW

    SW = 4
    s_pack = jnp.concatenate(
        [gd, vel, pitch.astype(jnp.float32)[..., None]], axis=-1
    ).reshape(B, SW * T)
    s_pack = jnp.pad(s_pack, ((0, 0), (0, _LANES)))
    SLEN = SW * T + _LANES

    mesh = plsc.VectorSubcoreMesh(core_axis_name="c", subcore_axis_name="s")

    @functools.partial(
        pl.kernel,
        out_type=jax.ShapeDtypeStruct((B, T, D), jnp.float32),
        mesh=mesh,
        scratch_types=[
            pltpu.VMEM((T, D), jnp.float32),
            pltpu.VMEM((NP, D_P), jnp.float32),
            pltpu.VMEM((2, D_GD), jnp.float32),
            pltpu.VMEM((1, D_V), jnp.float32),
            [pltpu.VMEM((SLEN,), jnp.float32)] * _NBUF,
            pltpu.VMEM((_NBUF, T, D), jnp.float32),
            [pltpu.SemaphoreType.DMA] * _NBUF,
        ],
    )
    def run(s_hbm, p_hbm, a_hbm, wgd_hbm, wv_hbm,
            out_hbm, a_v, p_v, wgd_v, wv_v, s_bufs, out_v, sems):
        wid = lax.axis_index("s") * _NC + lax.axis_index("c")
        pltpu.sync_copy(a_hbm, a_v)
        pltpu.sync_copy(p_hbm, p_v)
        pltpu.sync_copy(wgd_hbm, wgd_v)
        pltpu.sync_copy(wv_hbm, wv_v)

        wg = [[wgd_v[k, pl.ds(_LANES * q, _LANES)]
               for q in range(D_GD // _LANES)] for k in range(2)]
        wv = [wv_v[0, pl.ds(_LANES * q, _LANES)]
              for q in range(D_V // _LANES)]

        def compute_row(p):
            @plsc.parallel_loop(0, T, step=1, unroll=8)
            def tok_body(t):
                srow = s_bufs[p][pl.ds(SW * t, _LANES)]
                gd0 = srow[0]
                gd1 = srow[1]
                vl = srow[2]
                r = srow[3].astype(jnp.int32)
                for q in range(D_P // _LANES):
                    col = _LANES * q
                    out_v[p, t, pl.ds(col, _LANES)] = (
                        p_v[r, pl.ds(col, _LANES)]
                        + a_v[t, pl.ds(col, _LANES)])
                for q in range(D_GD // _LANES):
                    col = D_P + _LANES * q
                    out_v[p, t, pl.ds(col, _LANES)] = (
                        a_v[t, pl.ds(col, _LANES)]
                        + gd0 * wg[0][q] + gd1 * wg[1][q])
                for q in range(D_V // _LANES):
                    col = D_P + D_GD + _LANES * q
                    out_v[p, t, pl.ds(col, _LANES)] = (
                        a_v[t, pl.ds(col, _LANES)] + vl * wv[q])

        b0 = wid * RPW

        def row_pair(i2, c):
            for k in range(_NBUF):
                b = b0 + _NBUF * i2 + k

                @pl.when(i2 > 0)
                def _drain():
                    pltpu.make_async_copy(
                        out_v.at[k], out_hbm.at[b], sems[k]).wait()

                pltpu.sync_copy(s_hbm.at[b], s_bufs[k])
                compute_row(k)
                pltpu.async_copy(out_v.at[k], out_hbm.at[b], sems[k])
            return c

        lax.fori_loop(0, RPW // _NBUF, row_pair, 0)
        for k in range(_NBUF):
            pltpu.make_async_copy(
                out_v.at[k], out_hbm.at[b0], sems[k]).wait()

    return run(s_pack, P_table, A, Wgd, Wv.reshape(1, D_V))


# speedup vs baseline: 1.2807x; 1.2807x over previous
import functools

import jax
import jax.numpy as jnp
from jax import lax
from jax.experimental import pallas as pl
from jax.experimental.pallas import tpu as pltpu
from jax.experimental.pallas import tpu_sc as plsc

_NC = 2
_NS = 16
_LANES = 16
_NBUF = 2


def kernel(pitch, gd, vel, P_table, Wgd, bgd, Wv, bv, pos_table):
    B, T = pitch.shape
    NP = P_table.shape[0]
    D_P = P_table.shape[1]
    D_GD = Wgd.shape[1]
    D_V = Wv.shape[1]
    D = D_P + D_GD + D_V

    A = jnp.concatenate(
        [
            pos_table[:, :D_P],
            pos_table[:, D_P:] + bgd[None, :],
            jnp.broadcast_to(bv[None, :], (T, D_V)),
        ],
        axis=1,
    )

    NW = _NC * _NS
    RPW = B // NW

    SW = 4
    s_pack = jnp.concatenate(
        [gd, vel, pitch.astype(jnp.float32)[..., None]], axis=-1
    ).reshape(B, SW * T)
    s_pack = jnp.pad(s_pack, ((0, 0), (0, _LANES)))
    SLEN = SW * T + _LANES

    mesh = plsc.VectorSubcoreMesh(core_axis_name="c", subcore_axis_name="s")

    @functools.partial(
        pl.kernel,
        out_type=jax.ShapeDtypeStruct((B, T, D), jnp.float32),
        mesh=mesh,
        scratch_types=[
            pltpu.VMEM((T, D), jnp.float32),
            pltpu.VMEM((NP, D_P), jnp.float32),
            pltpu.VMEM((2, D_GD), jnp.float32),
            pltpu.VMEM((1, D_V), jnp.float32),
            [pltpu.VMEM((SLEN,), jnp.float32)] * _NBUF,
            pltpu.VMEM((_NBUF, T, D), jnp.float32),
            [pltpu.SemaphoreType.DMA] * _NBUF,
            [pltpu.SemaphoreType.DMA] * _NBUF,
        ],
    )
    def run(s_hbm, p_hbm, a_hbm, wgd_hbm, wv_hbm,
            out_hbm, a_v, p_v, wgd_v, wv_v, s_bufs, out_v, sems, s_sems):
        wid = lax.axis_index("s") * _NC + lax.axis_index("c")
        pltpu.sync_copy(a_hbm, a_v)
        pltpu.sync_copy(p_hbm, p_v)
        pltpu.sync_copy(wgd_hbm, wgd_v)
        pltpu.sync_copy(wv_hbm, wv_v)

        wg = [[wgd_v[k, pl.ds(_LANES * q, _LANES)]
               for q in range(D_GD // _LANES)] for k in range(2)]
        wv = [wv_v[0, pl.ds(_LANES * q, _LANES)]
              for q in range(D_V // _LANES)]

        def compute_row(p):
            @plsc.parallel_loop(0, T, step=1, unroll=4)
            def tok_body(t):
                srow = s_bufs[p][pl.ds(SW * t, _LANES)]
                gd0 = srow[0]
                gd1 = srow[1]
                vl = srow[2]
                r = srow[3].astype(jnp.int32)
                for q in range(D_P // _LANES):
                    col = _LANES * q
                    out_v[p, t, pl.ds(col, _LANES)] = (
                        p_v[r, pl.ds(col, _LANES)]
                        + a_v[t, pl.ds(col, _LANES)])
                for q in range(D_GD // _LANES):
                    col = D_P + _LANES * q
                    out_v[p, t, pl.ds(col, _LANES)] = (
                        a_v[t, pl.ds(col, _LANES)]
                        + gd0 * wg[0][q] + gd1 * wg[1][q])
                for q in range(D_V // _LANES):
                    col = D_P + D_GD + _LANES * q
                    out_v[p, t, pl.ds(col, _LANES)] = (
                        a_v[t, pl.ds(col, _LANES)] + vl * wv[q])

        b0 = wid * RPW
        pending_out = [None] * _NBUF
        pending_s = [None] * _NBUF
        pending_s[0] = pltpu.async_copy(s_hbm.at[b0], s_bufs[0], s_sems[0])
        for i in range(RPW):
            p = i % _NBUF
            pending_s[p].wait()
            if i + 1 < RPW:
                q = (i + 1) % _NBUF
                pending_s[q] = pltpu.async_copy(
                    s_hbm.at[b0 + i + 1], s_bufs[q], s_sems[q])
            if pending_out[p] is not None:
                pending_out[p].wait()
            compute_row(p)
            pending_out[p] = pltpu.async_copy(
                out_v.at[p], out_hbm.at[b0 + i], sems[p])
        for p in range(_NBUF):
            if pending_out[p] is not None:
                pending_out[p].wait()

    return run(s_pack, P_table, A, Wgd, Wv.reshape(1, D_V))


# speedup vs baseline: 1.3272x; 1.0364x over previous
import functools

import jax
import jax.numpy as jnp
from jax import lax
from jax.experimental import pallas as pl
from jax.experimental.pallas import tpu as pltpu
from jax.experimental.pallas import tpu_sc as plsc

_NC = 2
_NS = 16
_LANES = 16
_NBUF = 2


def kernel(pitch, gd, vel, P_table, Wgd, bgd, Wv, bv, pos_table):
    B, T = pitch.shape
    NP = P_table.shape[0]
    D_P = P_table.shape[1]
    D_GD = Wgd.shape[1]
    D_V = Wv.shape[1]
    D_POS = pos_table.shape[1]
    D = D_P + D_GD + D_V

    NW = _NC * _NS
    RPW = B // NW

    SW = 4
    s_pack = jnp.concatenate(
        [gd, vel, pitch.astype(jnp.float32)[..., None]], axis=-1
    ).reshape(B, SW * T)
    SLEN = SW * T

    T_MAIN = (SLEN - _LANES) // SW
    T_MAIN -= T_MAIN % 4

    mesh = plsc.VectorSubcoreMesh(core_axis_name="c", subcore_axis_name="s")

    @functools.partial(
        pl.kernel,
        out_type=jax.ShapeDtypeStruct((B, T, D), jnp.float32),
        mesh=mesh,
        scratch_types=[
            pltpu.VMEM((T, D_POS), jnp.float32),
            pltpu.VMEM((NP, D_P), jnp.float32),
            pltpu.VMEM((2, D_GD), jnp.float32),
            pltpu.VMEM((1, D_V), jnp.float32),
            pltpu.VMEM((D_GD,), jnp.float32),
            pltpu.VMEM((D_V,), jnp.float32),
            [pltpu.VMEM((SLEN,), jnp.float32)] * _NBUF,
            pltpu.VMEM((_NBUF, T, D), jnp.float32),
            [pltpu.SemaphoreType.DMA] * _NBUF,
            [pltpu.SemaphoreType.DMA] * _NBUF,
        ],
    )
    def run(s_hbm, p_hbm, pos_hbm, wgd_hbm, wv_hbm, bgd_hbm, bv_hbm,
            out_hbm, pos_v, p_v, wgd_v, wv_v, bgd_v, bv_v, s_bufs, out_v,
            sems, s_sems):
        wid = lax.axis_index("s") * _NC + lax.axis_index("c")
        pltpu.sync_copy(pos_hbm, pos_v)
        pltpu.sync_copy(p_hbm, p_v)
        pltpu.sync_copy(wgd_hbm, wgd_v)
        pltpu.sync_copy(wv_hbm, wv_v)
        pltpu.sync_copy(bgd_hbm, bgd_v)
        pltpu.sync_copy(bv_hbm, bv_v)

        NQG = D_GD // _LANES
        NQV = D_V // _LANES
        wg = [[wgd_v[k, pl.ds(_LANES * q, _LANES)] for q in range(NQG)]
              for k in range(2)]
        wv = [wv_v[0, pl.ds(_LANES * q, _LANES)] for q in range(NQV)]
        bg = [bgd_v[pl.ds(_LANES * q, _LANES)] for q in range(NQG)]
        bvv = [bv_v[pl.ds(_LANES * q, _LANES)] for q in range(NQV)]

        def emit_token(p, t, gd0, gd1, vl, r):
            for q in range(D_P // _LANES):
                col = _LANES * q
                out_v[p, t, pl.ds(col, _LANES)] = (
                    p_v[r, pl.ds(col, _LANES)]
                    + pos_v[t, pl.ds(col, _LANES)])
            for q in range(NQG):
                col = D_P + _LANES * q
                out_v[p, t, pl.ds(col, _LANES)] = (
                    pos_v[t, pl.ds(col, _LANES)]
                    + (gd0 * wg[0][q] + gd1 * wg[1][q] + bg[q]))
            for q in range(NQV):
                col = D_P + D_GD + _LANES * q
                out_v[p, t, pl.ds(col, _LANES)] = vl * wv[q] + bvv[q]

        def compute_row(p):
            @plsc.parallel_loop(0, T_MAIN, step=1, unroll=4)
            def tok_body(t):
                srow = s_bufs[p][pl.ds(SW * t, _LANES)]
                emit_token(p, t, srow[0], srow[1], srow[2],
                           srow[3].astype(jnp.int32))

            tail = s_bufs[p][pl.ds(SLEN - _LANES, _LANES)]
            for t in range(T_MAIN, T):
                off = SW * t - (SLEN - _LANES)
                emit_token(p, t, tail[off], tail[off + 1], tail[off + 2],
                           tail[off + 3].astype(jnp.int32))

        b0 = wid * RPW
        pending_out = [None] * _NBUF
        pending_s = [None] * _NBUF
        pending_s[0] = pltpu.async_copy(s_hbm.at[b0], s_bufs[0], s_sems[0])
        for i in range(RPW):
            p = i % _NBUF
            pending_s[p].wait()
            if i + 1 < RPW:
                q = (i + 1) % _NBUF
                pending_s[q] = pltpu.async_copy(
                    s_hbm.at[b0 + i + 1], s_bufs[q], s_sems[q])
            if pending_out[p] is not None:
                pending_out[p].wait()
            compute_row(p)
            pending_out[p] = pltpu.async_copy(
                out_v.at[p], out_hbm.at[b0 + i], sems[p])
        for p in range(_NBUF):
            if pending_out[p] is not None:
                pending_out[p].wait()

    return run(s_pack, P_table, pos_table, Wgd, Wv.reshape(1, D_V),
               bgd, bv)


# speedup vs baseline: 1.3731x; 1.0345x over previous
import functools

import jax
import jax.numpy as jnp
from jax import lax
from jax.experimental import pallas as pl
from jax.experimental.pallas import tpu as pltpu
from jax.experimental.pallas import tpu_sc as plsc

_NC = 2
_NS = 16
_LANES = 16
_NBUF = 3


def kernel(pitch, gd, vel, P_table, Wgd, bgd, Wv, bv, pos_table):
    B, T = pitch.shape
    NP = P_table.shape[0]
    D_P = P_table.shape[1]
    D_GD = Wgd.shape[1]
    D_V = Wv.shape[1]
    D_POS = pos_table.shape[1]
    D = D_P + D_GD + D_V

    NW = _NC * _NS
    RPW = B // NW

    SW = 4
    s_pack = jnp.concatenate(
        [gd, vel, pitch.astype(jnp.float32)[..., None]], axis=-1
    ).reshape(B, SW * T)
    SLEN = SW * T

    T_MAIN = (SLEN - _LANES) // SW
    T_MAIN -= T_MAIN % 4

    mesh = plsc.VectorSubcoreMesh(core_axis_name="c", subcore_axis_name="s")

    @functools.partial(
        pl.kernel,
        out_type=jax.ShapeDtypeStruct((B, T, D), jnp.float32),
        mesh=mesh,
        scratch_types=[
            pltpu.VMEM((T, D_POS), jnp.float32),
            pltpu.VMEM((NP, D_P), jnp.float32),
            pltpu.VMEM((2, D_GD), jnp.float32),
            pltpu.VMEM((1, D_V), jnp.float32),
            pltpu.VMEM((D_GD,), jnp.float32),
            pltpu.VMEM((D_V,), jnp.float32),
            [pltpu.VMEM((SLEN,), jnp.float32)] * _NBUF,
            pltpu.VMEM((_NBUF, T, D), jnp.float32),
            [pltpu.SemaphoreType.DMA] * _NBUF,
            [pltpu.SemaphoreType.DMA] * _NBUF,
            pltpu.SemaphoreType.DMA,
        ],
    )
    def run(s_hbm, p_hbm, pos_hbm, wgd_hbm, wv_hbm, bgd_hbm, bv_hbm,
            out_hbm, pos_v, p_v, wgd_v, wv_v, bgd_v, bv_v, s_bufs, out_v,
            sems, s_sems, c_sem):
        wid = lax.axis_index("s") * _NC + lax.axis_index("c")
        staging = [
            pltpu.async_copy(pos_hbm, pos_v, c_sem),
            pltpu.async_copy(p_hbm, p_v, c_sem),
            pltpu.async_copy(wgd_hbm, wgd_v, c_sem),
            pltpu.async_copy(wv_hbm, wv_v, c_sem),
            pltpu.async_copy(bgd_hbm, bgd_v, c_sem),
            pltpu.async_copy(bv_hbm, bv_v, c_sem),
        ]
        for cp in staging:
            cp.wait()

        NQG = D_GD // _LANES
        NQV = D_V // _LANES
        wg = [[wgd_v[k, pl.ds(_LANES * q, _LANES)] for q in range(NQG)]
              for k in range(2)]
        wv = [wv_v[0, pl.ds(_LANES * q, _LANES)] for q in range(NQV)]
        bg = [bgd_v[pl.ds(_LANES * q, _LANES)] for q in range(NQG)]
        bvv = [bv_v[pl.ds(_LANES * q, _LANES)] for q in range(NQV)]

        def emit_token(p, t, gd0, gd1, vl, r):
            for q in range(D_P // _LANES):
                col = _LANES * q
                out_v[p, t, pl.ds(col, _LANES)] = (
                    p_v[r, pl.ds(col, _LANES)]
                    + pos_v[t, pl.ds(col, _LANES)])
            for q in range(NQG):
                col = D_P + _LANES * q
                out_v[p, t, pl.ds(col, _LANES)] = (
                    pos_v[t, pl.ds(col, _LANES)]
                    + (gd0 * wg[0][q] + gd1 * wg[1][q] + bg[q]))
            for q in range(NQV):
                col = D_P + D_GD + _LANES * q
                out_v[p, t, pl.ds(col, _LANES)] = vl * wv[q] + bvv[q]

        def compute_row(p):
            @plsc.parallel_loop(0, T_MAIN, step=1, unroll=4)
            def tok_body(t):
                srow = s_bufs[p][pl.ds(SW * t, _LANES)]
                emit_token(p, t, srow[0], srow[1], srow[2],
                           srow[3].astype(jnp.int32))

            tail = s_bufs[p][pl.ds(SLEN - _LANES, _LANES)]
            for t in range(T_MAIN, T):
                off = SW * t - (SLEN - _LANES)
                emit_token(p, t, tail[off], tail[off + 1], tail[off + 2],
                           tail[off + 3].astype(jnp.int32))

        b0 = wid * RPW
        pending_out = [None] * _NBUF
        pending_s = [None] * _NBUF
        pending_s[0] = pltpu.async_copy(s_hbm.at[b0], s_bufs[0], s_sems[0])
        for i in range(RPW):
            p = i % _NBUF
            pending_s[p].wait()
            if i + 1 < RPW:
                q = (i + 1) % _NBUF
                pending_s[q] = pltpu.async_copy(
                    s_hbm.at[b0 + i + 1], s_bufs[q], s_sems[q])
            if pending_out[p] is not None:
                pending_out[p].wait()
            compute_row(p)
            pending_out[p] = pltpu.async_copy(
                out_v.at[p], out_hbm.at[b0 + i], sems[p])
        for p in range(_NBUF):
            if pending_out[p] is not None:
                pending_out[p].wait()

    return run(s_pack, P_table, pos_table, Wgd, Wv.reshape(1, D_V),
               bgd, bv)
